# baseline (speedup 1.0000x reference)
import functools
import jax
import jax.numpy as jnp
from jax import lax
from jax.experimental import pallas as pl
from jax.experimental.pallas import tpu as pltpu
from jax.experimental.pallas import tpu_sc as plsc

B_TOK = 4096
SEQ = 200
EMB = 64
SCALE = 8.0

NC = 2
NS = 16
NW = NC * NS
GROUP = 128
N_PER_W = (B_TOK * SEQ) // NW
N_GROUPS = N_PER_W // GROUP

_mesh = plsc.VectorSubcoreMesh(core_axis_name="c", subcore_axis_name="s")


@functools.partial(
    pl.kernel,
    mesh=_mesh,
    out_type=jax.ShapeDtypeStruct((B_TOK * SEQ, EMB), jnp.float32),
    scratch_types=[
        pltpu.VMEM((N_GROUPS, GROUP), jnp.int32),
        pltpu.VMEM((GROUP, EMB), jnp.float32),
        pltpu.SemaphoreType.DMA,
    ],
    compiler_params=pltpu.CompilerParams(use_tc_tiling_on_sc=False),
)
def _emb_lookup(tok_hbm, table_hbm, out_hbm, idx_v, buf, sem):
    wid = lax.axis_index("s") * NC + lax.axis_index("c")
    row0 = wid * N_GROUPS
    pltpu.sync_copy(tok_hbm.at[pl.ds(row0, N_GROUPS)], idx_v)

    def group_body(j, carry):
        pltpu.async_copy(table_hbm.at[idx_v.at[j]], buf, sem).wait()

        def scale_body(i, c):
            for q in range(EMB // 16):
                sl = pl.ds(q * 16, 16)
                buf[i, sl] = buf[i, sl] * SCALE
            return c

        lax.fori_loop(0, GROUP, scale_body, 0)
        pltpu.sync_copy(buf, out_hbm.at[pl.ds((row0 + j) * GROUP, GROUP)])
        return carry

    lax.fori_loop(0, N_GROUPS, group_body, 0)


def kernel(tokens, table):
    tok = tokens.reshape((B_TOK * SEQ) // GROUP, GROUP).astype(jnp.int32)
    out = _emb_lookup(tok, table)
    return out.reshape(B_TOK, SEQ, EMB)


# speedup vs baseline: 1.2074x; 1.2074x over previous
import functools
import jax
import jax.numpy as jnp
from jax import lax
from jax.experimental import pallas as pl
from jax.experimental.pallas import tpu as pltpu
from jax.experimental.pallas import tpu_sc as plsc

B_TOK = 4096
SEQ = 200
EMB = 64
SCALE = 8.0

NC = 2
NS = 16
NW = NC * NS
GROUP = 128
N_PER_W = (B_TOK * SEQ) // NW
N_GROUPS = N_PER_W // GROUP
NBUF = 4

_mesh = plsc.VectorSubcoreMesh(core_axis_name="c", subcore_axis_name="s")


@functools.partial(
    pl.kernel,
    mesh=_mesh,
    out_type=jax.ShapeDtypeStruct((B_TOK * SEQ, EMB), jnp.float32),
    scratch_types=[
        pltpu.VMEM((N_GROUPS, GROUP), jnp.int32),
        pltpu.VMEM((NBUF, GROUP, EMB), jnp.float32),
        pltpu.VMEM((NBUF, GROUP, EMB), jnp.float32),
    ]
    + [pltpu.SemaphoreType.DMA] * (2 * NBUF),
    compiler_params=pltpu.CompilerParams(use_tc_tiling_on_sc=False),
)
def _emb_lookup(tok_hbm, table_hbm, out_hbm, idx_v, gbuf, sbuf, *sems):
    gsem = sems[:NBUF]
    ssem = sems[NBUF:]
    wid = lax.axis_index("s") * NC + lax.axis_index("c")
    row0 = wid * N_GROUPS
    pltpu.sync_copy(tok_hbm.at[pl.ds(row0, N_GROUPS)], idx_v)

    def start_gather(g, b):
        pltpu.async_copy(table_hbm.at[idx_v.at[g]], gbuf.at[b], gsem[b])

    def start_store(g, b):
        pltpu.async_copy(sbuf.at[b], out_hbm.at[pl.ds((row0 + g) * GROUP, GROUP)],
                         ssem[b])

    def scale(b):
        def scale_row(i, c):
            for q in range(EMB // 16):
                sl = pl.ds(q * 16, 16)
                sbuf[b, i, sl] = gbuf[b, i, sl] * SCALE
            return c

        lax.fori_loop(0, GROUP, scale_row, 0)

    for b in range(NBUF):
        start_gather(b, b)
    for b in range(NBUF):
        pltpu.make_async_copy(table_hbm.at[idx_v.at[b]], gbuf.at[b],
                              gsem[b]).wait()
        scale(b)
        start_gather(b + NBUF, b)
        start_store(b, b)

    def body(t, carry):
        for b in range(NBUF):
            g = t * NBUF + b
            pltpu.make_async_copy(table_hbm.at[idx_v.at[g]], gbuf.at[b],
                                  gsem[b]).wait()
            pltpu.make_async_copy(
                sbuf.at[b], out_hbm.at[pl.ds((row0 + g) * GROUP, GROUP)],
                ssem[b]).wait()
            scale(b)

            @pl.when(g + NBUF < N_GROUPS)
            def _():
                start_gather(g + NBUF, b)

            start_store(g, b)
        return carry

    lax.fori_loop(1, N_GROUPS // NBUF, body, 0)

    for b in range(NBUF):
        g = N_GROUPS - NBUF + b
        pltpu.make_async_copy(
            sbuf.at[b], out_hbm.at[pl.ds((row0 + g) * GROUP, GROUP)],
            ssem[b]).wait()


def kernel(tokens, table):
    tok = tokens.reshape((B_TOK * SEQ) // GROUP, GROUP).astype(jnp.int32)
    out = _emb_lookup(tok, table)
    return out.reshape(B_TOK, SEQ, EMB)


# speedup vs baseline: 1.2080x; 1.0005x over previous
import functools
import jax
import jax.numpy as jnp
from jax import lax
from jax.experimental import pallas as pl
from jax.experimental.pallas import tpu as pltpu
from jax.experimental.pallas import tpu_sc as plsc

B_TOK = 4096
SEQ = 200
EMB = 64
SCALE = 8.0

NC = 2
NS = 16
NW = NC * NS
ROWS_PER_W = B_TOK // NW
G0 = 128
G1 = SEQ - G0
NBUF = 4

_mesh = plsc.VectorSubcoreMesh(core_axis_name="c", subcore_axis_name="s")


@functools.partial(
    pl.kernel,
    mesh=_mesh,
    out_type=jax.ShapeDtypeStruct((B_TOK, SEQ, EMB), jnp.float32),
    scratch_types=[
        pltpu.VMEM((ROWS_PER_W, SEQ), jnp.int32),
        pltpu.VMEM((NBUF, SEQ, EMB), jnp.float32),
        pltpu.VMEM((NBUF, SEQ, EMB), jnp.float32),
    ]
    + [pltpu.SemaphoreType.DMA] * (2 * NBUF),
    compiler_params=pltpu.CompilerParams(use_tc_tiling_on_sc=False),
)
def _emb_lookup(tok_hbm, table_hbm, out_hbm, idx_v, gbuf, sbuf, *sems):
    gsem = sems[:NBUF]
    ssem = sems[NBUF:]
    wid = lax.axis_index("s") * NC + lax.axis_index("c")
    row0 = wid * ROWS_PER_W
    pltpu.sync_copy(tok_hbm.at[pl.ds(row0, ROWS_PER_W)], idx_v)

    def start_gather(r, b):
        pltpu.async_copy(table_hbm.at[idx_v.at[r, pl.ds(0, G0)]],
                         gbuf.at[b, pl.ds(0, G0)], gsem[b])
        pltpu.async_copy(table_hbm.at[idx_v.at[r, pl.ds(G0, G1)]],
                         gbuf.at[b, pl.ds(G0, G1)], gsem[b])

    def wait_gather(r, b):
        pltpu.make_async_copy(table_hbm.at[idx_v.at[r, pl.ds(0, G0)]],
                              gbuf.at[b, pl.ds(0, G0)], gsem[b]).wait()
        pltpu.make_async_copy(table_hbm.at[idx_v.at[r, pl.ds(G0, G1)]],
                              gbuf.at[b, pl.ds(G0, G1)], gsem[b]).wait()

    def start_store(r, b):
        pltpu.async_copy(sbuf.at[b], out_hbm.at[row0 + r], ssem[b])

    def wait_store(r, b):
        pltpu.make_async_copy(sbuf.at[b], out_hbm.at[row0 + r],
                              ssem[b]).wait()

    def scale(b):
        def scale_row(i, c):
            for q in range(EMB // 16):
                sl = pl.ds(q * 16, 16)
                sbuf[b, i, sl] = gbuf[b, i, sl] * SCALE
            return c

        lax.fori_loop(0, SEQ, scale_row, 0)

    for b in range(NBUF):
        start_gather(b, b)
    for b in range(NBUF):
        wait_gather(b, b)
        scale(b)
        start_gather(b + NBUF, b)
        start_store(b, b)

    def body(t, carry):
        for b in range(NBUF):
            r = t * NBUF + b
            wait_gather(r, b)
            wait_store(r, b)
            scale(b)

            @pl.when(r + NBUF < ROWS_PER_W)
            def _():
                start_gather(r + NBUF, b)

            start_store(r, b)
        return carry

    lax.fori_loop(1, ROWS_PER_W // NBUF, body, 0)

    for b in range(NBUF):
        wait_store(ROWS_PER_W - NBUF + b, b)


def kernel(tokens, table):
    return _emb_lookup(tokens.astype(jnp.int32), table)


# speedup vs baseline: 1.3068x; 1.0818x over previous
import functools
import jax
import jax.numpy as jnp
from jax import lax
from jax.experimental import pallas as pl
from jax.experimental.pallas import tpu as pltpu
from jax.experimental.pallas import tpu_sc as plsc

B_TOK = 4096
SEQ = 200
EMB = 64
VOCAB_ROWS = 1000000
SCALE = 8.0

NC = 2
NS = 16
NW = NC * NS
BLK = B_TOK // NW
KB = EMB // 8
PITCH = BLK + 1
NBUF = 4
TCHUNK = 4096

_mesh = plsc.VectorSubcoreMesh(core_axis_name="c", subcore_axis_name="s")


@functools.partial(
    pl.kernel,
    mesh=_mesh,
    out_type=jax.ShapeDtypeStruct((SEQ, KB, NW, 8, BLK), jnp.float32),
    scratch_types=[
        pltpu.VMEM((SEQ, BLK), jnp.int32),
        pltpu.VMEM((NBUF, BLK, EMB), jnp.float32),
        pltpu.VMEM((NBUF, EMB, PITCH), jnp.float32),
    ]
    + [pltpu.SemaphoreType.DMA] * (2 * NBUF),
    compiler_params=pltpu.CompilerParams(
        use_tc_tiling_on_sc=False, needs_layout_passes=False),
)
def _emb_lookup(tok_hbm, table_hbm, out_hbm, idx_v, gbuf, sbuf, *sems):
    gsem = sems[:NBUF]
    ssem = sems[NBUF:]
    wid = lax.axis_index("s") * NC + lax.axis_index("c")
    pltpu.sync_copy(tok_hbm.at[:, wid], idx_v)
    iota16 = lax.iota(jnp.int32, 16)
    rows_q = [iota16 + q * 16 for q in range(EMB // 16)]

    def start_gather(s, b):
        pltpu.async_copy(table_hbm.at[idx_v.at[s]], gbuf.at[b], gsem[b])

    def wait_gather(s, b):
        pltpu.make_async_copy(table_hbm.at[idx_v.at[s]], gbuf.at[b],
                              gsem[b]).wait()

    def start_store(s, b):
        for kb in range(KB):
            pltpu.async_copy(sbuf.at[b, pl.ds(kb * 8, 8), pl.ds(0, BLK)],
                             out_hbm.at[s, kb, wid], ssem[b])

    def wait_store(s, b):
        for kb in range(KB):
            pltpu.make_async_copy(sbuf.at[b, pl.ds(kb * 8, 8), pl.ds(0, BLK)],
                                  out_hbm.at[s, kb, wid], ssem[b]).wait()

    def xform(b):
        def tok_body(tb, c):
            cols = iota16 * 0 + tb
            for q in range(EMB // 16):
                v = gbuf[b, tb, pl.ds(q * 16, 16)]
                plsc.store_scatter(sbuf.at[b], [rows_q[q], cols], v)
            return c

        lax.fori_loop(0, BLK, tok_body, 0)

    for b in range(NBUF):
        start_gather(b, b)
    for b in range(NBUF):
        wait_gather(b, b)
        xform(b)
        start_gather(b + NBUF, b)
        start_store(b, b)

    def body(t, carry):
        for b in range(NBUF):
            s = t * NBUF + b
            wait_gather(s, b)
            wait_store(s, b)
            xform(b)

            @pl.when(s + NBUF < SEQ)
            def _():
                start_gather(s + NBUF, b)

            start_store(s, b)
        return carry

    lax.fori_loop(1, SEQ // NBUF, body, 0)

    for b in range(NBUF):
        wait_store(SEQ - NBUF + b, b)


def _tc_scale_transpose(tt_ref, out_ref):
    out_ref[...] = jnp.transpose(tt_ref[...]) * SCALE


_scale_transpose = pl.pallas_call(
    _tc_scale_transpose,
    grid=(pl.cdiv(VOCAB_ROWS, TCHUNK),),
    in_specs=[pl.BlockSpec((EMB, TCHUNK), lambda i: (0, i))],
    out_specs=pl.BlockSpec((TCHUNK, EMB), lambda i: (i, 0)),
    out_shape=jax.ShapeDtypeStruct((VOCAB_ROWS, EMB), jnp.float32),
)


def kernel(tokens, table):
    tok_t = tokens.astype(jnp.int32).T.reshape(SEQ, NW, BLK)
    scaled = _scale_transpose(table.T)
    flat = _emb_lookup(tok_t, scaled)
    return flat.transpose(2, 4, 0, 1, 3).reshape(B_TOK, SEQ, EMB)
